# baseline (speedup 1.0000x reference)
import functools

import jax
import jax.numpy as jnp
from jax import lax
from jax.experimental import pallas as pl
from jax.experimental.pallas import tpu as pltpu
from jax.experimental.pallas import tpu_sc as plsc

N = 10000
E = 320000
IN_NF = 128
IN_EF = 16
H1 = 32
H2 = 32
OUT_NF = 128

NC, NS, LANES = 2, 16, 16
NW = NC * NS

NSLAB = 2
SE = E // NSLAB
EPW = SE // NW
GC = 128
NGC = EPW // GC
GTAIL = EPW - NGC * GC

SC_CHUNK = 2000
NSC = SE // SC_CHUNK
NGRP = SC_CHUNK // LANES

BE = 1280

def _mesh():
    return plsc.VectorSubcoreMesh(
        core_axis_name="c", subcore_axis_name="s",
        num_cores=NC, num_subcores=NS)


def _leaky(x):
    return jnp.where(x >= 0, x, 0.2 * x)


def _pq_body(nf_ref, ws_ref, wd_ref, p_ref, q_ref):
    nf = nf_ref[...]
    p_ref[...] = jnp.dot(nf, ws_ref[...],
                         preferred_element_type=jnp.float32).astype(jnp.bfloat16)
    q_ref[...] = jnp.dot(nf, wd_ref[...],
                         preferred_element_type=jnp.float32).astype(jnp.bfloat16)


def _pq_call(nf, w1s, w1d):
    return pl.pallas_call(
        _pq_body,
        out_shape=[jax.ShapeDtypeStruct((N, 64), jnp.bfloat16)] * 2,
    )(nf, w1s, w1d)


@functools.cache
def _gather_kernel_fn():
    return pl.kernel(
        _gather_body,
        out_type=[jax.ShapeDtypeStruct((SE, 64), jnp.bfloat16)] * 2,
        mesh=_mesh(),
        scratch_types=[
            pltpu.VMEM((GC,), jnp.int32),
            pltpu.VMEM((GC,), jnp.int32),
            pltpu.VMEM((GC, 64), jnp.bfloat16),
            pltpu.VMEM((GC, 64), jnp.bfloat16),
            pltpu.VMEM((GC,), jnp.int32),
            pltpu.VMEM((GC,), jnp.int32),
            pltpu.VMEM((GC, 64), jnp.bfloat16),
            pltpu.VMEM((GC, 64), jnp.bfloat16),
            pltpu.SemaphoreType.DMA,
            pltpu.SemaphoreType.DMA,
            pltpu.SemaphoreType.DMA,
            pltpu.SemaphoreType.DMA,
        ],
        compiler_params=pltpu.CompilerParams(use_tc_tiling_on_sc=False),
    )


def _gather_body(p_hbm, q_hbm, src_hbm, dst_hbm, ps_hbm, qd_hbm,
                 sidx_a, didx_a, prow_a, qrow_a,
                 sidx_b, didx_b, prow_b, qrow_b,
                 gsem_a, gsem_b, wsem_a, wsem_b):
    wid = lax.axis_index("s") * NC + lax.axis_index("c")
    base0 = wid * EPW
    A = (sidx_a, didx_a, prow_a, qrow_a, gsem_a, wsem_a)
    B = (sidx_b, didx_b, prow_b, qrow_b, gsem_b, wsem_b)

    def load_and_gather(base, bufs):
        sidx, didx, prow, qrow, gsem, _ = bufs
        pltpu.sync_copy(src_hbm.at[pl.ds(base, GC)], sidx)
        pltpu.sync_copy(dst_hbm.at[pl.ds(base, GC)], didx)
        pltpu.async_copy(p_hbm.at[sidx], prow, gsem)
        pltpu.async_copy(q_hbm.at[didx], qrow, gsem)

    def finish_and_write(base, bufs):
        sidx, didx, prow, qrow, gsem, wsem = bufs
        pltpu.make_async_copy(p_hbm.at[sidx], prow, gsem).wait()
        pltpu.make_async_copy(q_hbm.at[didx], qrow, gsem).wait()
        pltpu.async_copy(prow, ps_hbm.at[pl.ds(base, GC)], wsem)
        pltpu.async_copy(qrow, qd_hbm.at[pl.ds(base, GC)], wsem)

    def drain_writes(bufs):
        _, _, prow, qrow, _, wsem = bufs
        pltpu.make_async_copy(prow, ps_hbm.at[pl.ds(0, GC)], wsem).wait()
        pltpu.make_async_copy(qrow, qd_hbm.at[pl.ds(0, GC)], wsem).wait()

    def pair(ci, carry):
        c0 = base0 + (2 * ci) * GC
        c1 = c0 + GC

        @pl.when(ci > 0)
        def _():
            drain_writes(A)
            drain_writes(B)

        load_and_gather(c0, A)
        load_and_gather(c1, B)
        finish_and_write(c0, A)
        finish_and_write(c1, B)
        return carry

    lax.fori_loop(0, NGC // 2, pair, 0)
    drain_writes(A)
    drain_writes(B)

    def do_chunk(base, n):
        sidx, didx, prow, qrow, gsem, _ = A
        pltpu.sync_copy(src_hbm.at[pl.ds(base, n)], sidx.at[pl.ds(0, n)])
        pltpu.sync_copy(dst_hbm.at[pl.ds(base, n)], didx.at[pl.ds(0, n)])
        cp1 = pltpu.async_copy(p_hbm.at[sidx.at[pl.ds(0, n)]],
                               prow.at[pl.ds(0, n)], gsem)
        cp2 = pltpu.async_copy(q_hbm.at[didx.at[pl.ds(0, n)]],
                               qrow.at[pl.ds(0, n)], gsem)
        cp1.wait()
        cp2.wait()
        pltpu.sync_copy(prow.at[pl.ds(0, n)], ps_hbm.at[pl.ds(base, n)])
        pltpu.sync_copy(qrow.at[pl.ds(0, n)], qd_hbm.at[pl.ds(base, n)])

    if NGC % 2:
        do_chunk(base0 + (NGC - 1) * GC, GC)
    if GTAIL:
        do_chunk(base0 + NGC * GC, GTAIL)


def _edge_mlp_body(ps_ref, qd_ref, ef_ref,
                   w1e_ref, b1_ref, w2_ref, b2_ref, w3_ref, b3_ref,
                   w4k_ref, b4k_ref, w4f_ref, b4f_ref,
                   out_ref):
    bf = jnp.bfloat16
    h1 = (ps_ref[...].astype(jnp.float32) + qd_ref[...].astype(jnp.float32))
    h1 += jnp.dot(ef_ref[...], w1e_ref[...], preferred_element_type=jnp.float32)
    h1 = _leaky(h1 + b1_ref[...]).astype(bf)
    h2 = _leaky(jnp.dot(h1, w2_ref[...].astype(bf),
                        preferred_element_type=jnp.float32)
                + b2_ref[...]).astype(bf)
    h3 = _leaky(jnp.dot(h2, w3_ref[...].astype(bf),
                        preferred_element_type=jnp.float32)
                + b3_ref[...]).astype(bf)
    kT = lax.dot_general(w4k_ref[...].astype(bf), h3, (((0,), (1,)), ((), ())),
                         preferred_element_type=jnp.float32)
    kT = 1.0 / (1.0 + jnp.exp(-(kT + b4k_ref[...])))
    fT = lax.dot_general(w4f_ref[...].astype(bf), h3, (((0,), (1,)), ((), ())),
                         preferred_element_type=jnp.float32)
    out_ref[...] = (fT + b4f_ref[...]) * kT


def _edge_mlp_call(ps, qd, ef, w1e, b1, w2, b2, w3, b3, w4k, b4k, w4f, b4f):
    grid = SE // BE
    wspec = lambda shape: pl.BlockSpec(shape, lambda i: (0,) * len(shape))
    return pl.pallas_call(
        _edge_mlp_body,
        grid=(grid,),
        in_specs=[
            pl.BlockSpec((BE, 64), lambda i: (i, 0)),
            pl.BlockSpec((BE, 64), lambda i: (i, 0)),
            pl.BlockSpec((BE, IN_EF), lambda i: (i, 0)),
            wspec((IN_EF, 64)), wspec((1, 64)),
            wspec((64, 128)), wspec((1, 128)),
            wspec((128, 64)), wspec((1, 64)),
            wspec((64, 1)), wspec((1, 1)),
            wspec((64, 64)), wspec((64, 1)),
        ],
        out_specs=pl.BlockSpec((64, BE), lambda i: (0, i)),
        out_shape=jax.ShapeDtypeStruct((64, SE), jnp.float32),
    )(ps, qd, ef, w1e, b1, w2, b2, w3, b3, w4k, b4k, w4f, b4f)


@functools.cache
def _scatter_kernel_fn():
    return pl.kernel(
        _scatter_body,
        out_type=jax.ShapeDtypeStruct((64 * N,), jnp.float32),
        mesh=_mesh(),
        scratch_types=[
            pltpu.VMEM((N,), jnp.float32),
            pltpu.VMEM((N,), jnp.float32),
            pltpu.VMEM((N,), jnp.float32),
            pltpu.VMEM((N,), jnp.float32),
            pltpu.VMEM((SC_CHUNK,), jnp.int32),
            pltpu.VMEM((SC_CHUNK,), jnp.float32),
            pltpu.VMEM((SC_CHUNK,), jnp.float32),
            pltpu.VMEM((SC_CHUNK,), jnp.int32),
            pltpu.VMEM((SC_CHUNK,), jnp.float32),
            pltpu.VMEM((SC_CHUNK,), jnp.float32),
            pltpu.SemaphoreType.DMA,
            pltpu.SemaphoreType.DMA,
        ],
        compiler_params=pltpu.CompilerParams(needs_layout_passes=False),
    )


def _scatter_body(eft_hbm, dst_hbm, agg_hbm, sacc, macc, sacc1, macc1,
                  didx_a, v1_a, v2_a, didx_b, v1_b, v2_b, sem_a, sem_b):
    wid = lax.axis_index("s") * NC + lax.axis_index("c")

    def init_loop(i, carry):
        sl = pl.ds(i * LANES, LANES)
        sacc[sl] = jnp.zeros((LANES,), jnp.float32)
        sacc1[sl] = jnp.zeros((LANES,), jnp.float32)
        macc[sl] = jnp.full((LANES,), -jnp.inf, jnp.float32)
        macc1[sl] = jnp.full((LANES,), -jnp.inf, jnp.float32)
        return carry

    lax.fori_loop(0, N // LANES, init_loop, 0)

    sum_off = wid * SE
    max_off = (32 + wid) * SE

    def issue(base, bufs, sem):
        bd, b1, b2 = bufs
        pltpu.async_copy(dst_hbm.at[pl.ds(base, SC_CHUNK)], bd, sem)
        pltpu.async_copy(eft_hbm.at[pl.ds(sum_off + base, SC_CHUNK)], b1, sem)
        pltpu.async_copy(eft_hbm.at[pl.ds(max_off + base, SC_CHUNK)], b2, sem)

    def drain(bufs, sem):
        bd, b1, b2 = bufs
        pltpu.make_async_copy(dst_hbm.at[pl.ds(0, SC_CHUNK)], bd, sem).wait()
        pltpu.make_async_copy(eft_hbm.at[pl.ds(0, SC_CHUNK)], b1, sem).wait()
        pltpu.make_async_copy(eft_hbm.at[pl.ds(0, SC_CHUNK)], b2, sem).wait()

    def compute(bufs):
        bd, b1, b2 = bufs
        def one_group(off, fail, sa, ma):
            sl = pl.ds(off, LANES)
            d = bd[sl]
            s = b1[sl]
            m = b2[sl]
            plsc.addupdate_scatter(sa, [d], s)
            cur = plsc.load_gather(ma, [d])
            new = jnp.maximum(cur, m)
            plsc.store_scatter(ma, [d], new)
            cur2 = plsc.load_gather(ma, [d])
            plsc.store_scatter(ma, [d], jnp.maximum(cur2, new),
                              mask=cur2 < new)
            chk = plsc.load_gather(ma, [d])
            return fail | (chk < new)

        def grp(gi, fail):
            fail = one_group(gi * (2 * LANES), fail, sacc, macc)
            return one_group(gi * (2 * LANES) + LANES, fail, sacc1, macc1)

        fail = lax.fori_loop(0, NGRP // 2, grp,
                             jnp.zeros((LANES,), jnp.bool_))
        if NGRP % 2:
            fail = one_group((NGRP - 1) * LANES, fail, sacc, macc)

        @pl.when(jnp.any(fail))
        def _fixup():
            def fix_group(off, ma):
                sl = pl.ds(off, LANES)
                d = bd[sl]
                m = b2[sl]

                def retry_body(need):
                    c2 = plsc.load_gather(ma, [d])
                    plsc.store_scatter(ma, [d], jnp.maximum(c2, m),
                                      mask=need)
                    c3 = plsc.load_gather(ma, [d])
                    return c3 < m

                lax.while_loop(lambda n: jnp.any(n), retry_body,
                               plsc.load_gather(ma, [d]) < m)

            def grp2(gi, carry):
                fix_group(gi * (2 * LANES), macc)
                fix_group(gi * (2 * LANES) + LANES, macc1)
                return carry

            lax.fori_loop(0, NGRP // 2, grp2, 0)
            if NGRP % 2:
                fix_group((NGRP - 1) * LANES, macc)

    bufs_a = (didx_a, v1_a, v2_a)
    bufs_b = (didx_b, v1_b, v2_b)
    issue(0, bufs_a, sem_a)

    def pair_loop(ci, carry):
        base_a = (2 * ci) * SC_CHUNK
        base_b = base_a + SC_CHUNK
        base_n = jnp.minimum(base_a + 2 * SC_CHUNK, SE - SC_CHUNK)
        drain(bufs_a, sem_a)
        issue(base_b, bufs_b, sem_b)
        compute(bufs_a)
        drain(bufs_b, sem_b)
        issue(base_n, bufs_a, sem_a)
        compute(bufs_b)
        return carry

    lax.fori_loop(0, NSC // 2, pair_loop, 0)
    drain(bufs_a, sem_a)

    def merge_loop(i, carry):
        sl = pl.ds(i * LANES, LANES)
        sacc[sl] = sacc[sl] + sacc1[sl]
        macc[sl] = jnp.maximum(macc[sl], macc1[sl])
        return carry

    lax.fori_loop(0, N // LANES, merge_loop, 0)

    pltpu.sync_copy(sacc, agg_hbm.at[pl.ds(wid * N, N)])
    pltpu.sync_copy(macc, agg_hbm.at[pl.ds((32 + wid) * N, N)])


def _out_mlp_body(nf_ref, agg0_ref, agg1_ref,
                  wa_ref, wb_ref, wc_ref, b1_ref,
                  w2_ref, b2_ref, w3_ref, b3_ref, w4_ref, b4_ref,
                  out_ref):
    h1 = jnp.dot(nf_ref[...], wa_ref[...], preferred_element_type=jnp.float32)
    sumt = agg0_ref[:32, :] + agg1_ref[:32, :]
    h1 += lax.dot_general(sumt, wb_ref[...], (((0,), (0,)), ((), ())),
                          preferred_element_type=jnp.float32)
    mx = jnp.maximum(agg0_ref[32:, :], agg1_ref[32:, :])
    mx = jnp.where(jnp.isinf(mx) & (mx < 0), 0.0, mx)
    h1 += lax.dot_general(mx, wc_ref[...], (((0,), (0,)), ((), ())),
                          preferred_element_type=jnp.float32)
    h1 = _leaky(h1 + b1_ref[...])
    h2 = _leaky(jnp.dot(h1, w2_ref[...], preferred_element_type=jnp.float32)
                + b2_ref[...])
    h3 = _leaky(jnp.dot(h2, w3_ref[...], preferred_element_type=jnp.float32)
                + b3_ref[...])
    out_ref[...] = jnp.dot(h3, w4_ref[...],
                           preferred_element_type=jnp.float32) + b4_ref[...]


def _out_mlp_call(nf, agg0, agg1, wa, wb, wc, b1, w2, b2, w3, b3, w4, b4):
    return pl.pallas_call(
        _out_mlp_body,
        out_shape=jax.ShapeDtypeStruct((N, OUT_NF), jnp.float32),
    )(nf, agg0, agg1, wa, wb, wc, b1, w2, b2, w3, b3, w4, b4)


@jax.jit
def kernel(nf, ef, msg_params, red_params, edge_index):
    w1, b1, w2, b2, w3, b3, w4, b4 = msg_params
    wr1, br1, wr2, br2, wr3, br3, wr4, br4 = red_params

    src = edge_index[0].astype(jnp.int32)
    dst = edge_index[1].astype(jnp.int32)

    w1s, w1d, w1e = w1[:IN_NF], w1[IN_NF:2 * IN_NF], w1[2 * IN_NF:]
    w4k, w4f = w4[:, :1], w4[:, 1:]
    b4k, b4f = b4[:1].reshape(1, 1), b4[1:].reshape(64, 1)

    p, q = _pq_call(nf, w1s, w1d)
    srcs = [src[s * SE:(s + 1) * SE] for s in range(NSLAB)]
    dsts = [dst[s * SE:(s + 1) * SE] for s in range(NSLAB)]
    gathered = [_gather_kernel_fn()(p, q, srcs[s], dsts[s])
                for s in range(NSLAB)]
    efts = [_edge_mlp_call(gathered[s][0], gathered[s][1],
                           ef[s * SE:(s + 1) * SE],
                           w1e, b1.reshape(1, 64), w2, b2.reshape(1, 128),
                           w3, b3.reshape(1, 64), w4k, b4k, w4f, b4f)
            for s in range(NSLAB)]
    aggs = [_scatter_kernel_fn()(efts[s].reshape(64 * SE), dsts[s])
            for s in range(NSLAB)]

    wa, wb, wc = wr1[:IN_NF], wr1[IN_NF:IN_NF + 32], wr1[IN_NF + 32:]
    return _out_mlp_call(nf, aggs[0].reshape(64, N), aggs[1].reshape(64, N),
                         wa, wb, wc, br1.reshape(1, 64),
                         wr2, br2.reshape(1, 128), wr3, br3.reshape(1, 64),
                         wr4, br4.reshape(1, OUT_NF))


# speedup vs baseline: 1.2971x; 1.2971x over previous
import functools

import jax
import jax.numpy as jnp
from jax import lax
from jax.experimental import pallas as pl
from jax.experimental.pallas import tpu as pltpu
from jax.experimental.pallas import tpu_sc as plsc

N = 10000
E = 320000
IN_NF = 128
IN_EF = 16
H1 = 32
H2 = 32
OUT_NF = 128

NC, NS, LANES = 2, 16, 16
NW = NC * NS

NSLAB = 2
SE = E // NSLAB
EPW = SE // NW
GC = 128
NGC = EPW // GC
GTAIL = EPW - NGC * GC

SC_CHUNK = 2000
NSC = SE // SC_CHUNK
NGRP = SC_CHUNK // LANES

BE = 1280

def _mesh():
    return plsc.VectorSubcoreMesh(
        core_axis_name="c", subcore_axis_name="s",
        num_cores=NC, num_subcores=NS)


def _leaky(x):
    return jnp.where(x >= 0, x, 0.2 * x)


def _pq_body(nf_ref, ws_ref, wd_ref, p_ref, q_ref):
    nf = nf_ref[...]
    p_ref[...] = jnp.dot(nf, ws_ref[...], preferred_element_type=jnp.float32)
    q_ref[...] = jnp.dot(nf, wd_ref[...], preferred_element_type=jnp.float32)


def _pq_call(nf, w1s, w1d):
    return pl.pallas_call(
        _pq_body,
        out_shape=[jax.ShapeDtypeStruct((N, 64), jnp.float32)] * 2,
    )(nf, w1s, w1d)


@functools.cache
def _gather_kernel_fn():
    return pl.kernel(
        _gather_body,
        out_type=[jax.ShapeDtypeStruct((SE, 64), jnp.float32)] * 2,
        mesh=_mesh(),
        scratch_types=[
            pltpu.VMEM((GC,), jnp.int32),
            pltpu.VMEM((GC,), jnp.int32),
            pltpu.VMEM((GC, 64), jnp.float32),
            pltpu.VMEM((GC, 64), jnp.float32),
            pltpu.VMEM((GC,), jnp.int32),
            pltpu.VMEM((GC,), jnp.int32),
            pltpu.VMEM((GC, 64), jnp.float32),
            pltpu.VMEM((GC, 64), jnp.float32),
            pltpu.SemaphoreType.DMA,
            pltpu.SemaphoreType.DMA,
            pltpu.SemaphoreType.DMA,
            pltpu.SemaphoreType.DMA,
        ],
        compiler_params=pltpu.CompilerParams(use_tc_tiling_on_sc=False),
    )


def _gather_body(p_hbm, q_hbm, src_hbm, dst_hbm, ps_hbm, qd_hbm,
                 sidx_a, didx_a, prow_a, qrow_a,
                 sidx_b, didx_b, prow_b, qrow_b,
                 gsem_a, gsem_b, wsem_a, wsem_b):
    wid = lax.axis_index("s") * NC + lax.axis_index("c")
    base0 = wid * EPW
    A = (sidx_a, didx_a, prow_a, qrow_a, gsem_a, wsem_a)
    B = (sidx_b, didx_b, prow_b, qrow_b, gsem_b, wsem_b)

    def load_and_gather(base, bufs):
        sidx, didx, prow, qrow, gsem, _ = bufs
        pltpu.sync_copy(src_hbm.at[pl.ds(base, GC)], sidx)
        pltpu.sync_copy(dst_hbm.at[pl.ds(base, GC)], didx)
        pltpu.async_copy(p_hbm.at[sidx], prow, gsem)
        pltpu.async_copy(q_hbm.at[didx], qrow, gsem)

    def finish_and_write(base, bufs):
        sidx, didx, prow, qrow, gsem, wsem = bufs
        pltpu.make_async_copy(p_hbm.at[sidx], prow, gsem).wait()
        pltpu.make_async_copy(q_hbm.at[didx], qrow, gsem).wait()
        pltpu.async_copy(prow, ps_hbm.at[pl.ds(base, GC)], wsem)
        pltpu.async_copy(qrow, qd_hbm.at[pl.ds(base, GC)], wsem)

    def drain_writes(bufs):
        _, _, prow, qrow, _, wsem = bufs
        pltpu.make_async_copy(prow, ps_hbm.at[pl.ds(0, GC)], wsem).wait()
        pltpu.make_async_copy(qrow, qd_hbm.at[pl.ds(0, GC)], wsem).wait()

    def pair(ci, carry):
        c0 = base0 + (2 * ci) * GC
        c1 = c0 + GC

        @pl.when(ci > 0)
        def _():
            drain_writes(A)
            drain_writes(B)

        load_and_gather(c0, A)
        load_and_gather(c1, B)
        finish_and_write(c0, A)
        finish_and_write(c1, B)
        return carry

    lax.fori_loop(0, NGC // 2, pair, 0)
    drain_writes(A)
    drain_writes(B)

    def do_chunk(base, n):
        sidx, didx, prow, qrow, gsem, _ = A
        pltpu.sync_copy(src_hbm.at[pl.ds(base, n)], sidx.at[pl.ds(0, n)])
        pltpu.sync_copy(dst_hbm.at[pl.ds(base, n)], didx.at[pl.ds(0, n)])
        cp1 = pltpu.async_copy(p_hbm.at[sidx.at[pl.ds(0, n)]],
                               prow.at[pl.ds(0, n)], gsem)
        cp2 = pltpu.async_copy(q_hbm.at[didx.at[pl.ds(0, n)]],
                               qrow.at[pl.ds(0, n)], gsem)
        cp1.wait()
        cp2.wait()
        pltpu.sync_copy(prow.at[pl.ds(0, n)], ps_hbm.at[pl.ds(base, n)])
        pltpu.sync_copy(qrow.at[pl.ds(0, n)], qd_hbm.at[pl.ds(base, n)])

    if NGC % 2:
        do_chunk(base0 + (NGC - 1) * GC, GC)
    if GTAIL:
        do_chunk(base0 + NGC * GC, GTAIL)


def _edge_mlp_body(ps_ref, qd_ref, efp_ref,
                   w1e_ref, b1_ref, w2_ref, b2_ref, w3_ref, b3_ref,
                   w4k_ref, b4k_ref, w4f_ref, b4f_ref,
                   out_ref):
    x = ps_ref[...] + qd_ref[...]
    x += jnp.dot(efp_ref[...], w1e_ref[...], preferred_element_type=jnp.float32)
    h1 = _leaky(x + b1_ref[...])
    h2 = _leaky(jnp.dot(h1, w2_ref[...], preferred_element_type=jnp.float32)
                + b2_ref[...])
    h3 = _leaky(jnp.dot(h2, w3_ref[...], preferred_element_type=jnp.float32)
                + b3_ref[...])
    kT = lax.dot_general(w4k_ref[...], h3, (((0,), (1,)), ((), ())),
                         preferred_element_type=jnp.float32)
    kT = 1.0 / (1.0 + jnp.exp(-(kT + b4k_ref[...])))
    fT = lax.dot_general(w4f_ref[...], h3, (((0,), (1,)), ((), ())),
                         preferred_element_type=jnp.float32)
    fT = fT + b4f_ref[...]
    out_ref[0:64, :] = fT[0:64, :] * kT[0:1, :]
    out_ref[64:128, :] = fT[64:128, :] * kT[1:2, :]


BE2 = BE // 2


def _edge_mlp_call(ps, qd, efp, w1e, b1, w2, b2, w3, b3, w4k, b4k, w4f, b4f):
    grid = SE // BE
    wspec = lambda shape: pl.BlockSpec(shape, lambda i: (0,) * len(shape))
    return pl.pallas_call(
        _edge_mlp_body,
        grid=(grid,),
        in_specs=[
            pl.BlockSpec((BE2, 128), lambda i: (i, 0)),
            pl.BlockSpec((BE2, 128), lambda i: (i, 0)),
            pl.BlockSpec((BE2, 2 * IN_EF), lambda i: (i, 0)),
            wspec((2 * IN_EF, 128)), wspec((1, 128)),
            wspec((128, 256)), wspec((1, 256)),
            wspec((256, 128)), wspec((1, 128)),
            wspec((128, 2)), wspec((2, 1)),
            wspec((128, 128)), wspec((128, 1)),
        ],
        out_specs=pl.BlockSpec((128, BE2), lambda i: (0, i)),
        out_shape=jax.ShapeDtypeStruct((128, SE // 2), jnp.float32),
    )(ps.reshape(SE // 2, 128), qd.reshape(SE // 2, 128), efp,
      w1e, b1, w2, b2, w3, b3, w4k, b4k, w4f, b4f)


@functools.cache
def _scatter_kernel_fn():
    return pl.kernel(
        _scatter_body,
        out_type=jax.ShapeDtypeStruct((64 * N,), jnp.float32),
        mesh=_mesh(),
        scratch_types=[
            pltpu.VMEM((N,), jnp.float32),
            pltpu.VMEM((N,), jnp.float32),
            pltpu.VMEM((N,), jnp.float32),
            pltpu.VMEM((N,), jnp.float32),
            pltpu.VMEM((SC_CHUNK,), jnp.int32),
            pltpu.VMEM((SC_CHUNK,), jnp.float32),
            pltpu.VMEM((SC_CHUNK,), jnp.float32),
            pltpu.VMEM((SC_CHUNK,), jnp.int32),
            pltpu.VMEM((SC_CHUNK,), jnp.float32),
            pltpu.VMEM((SC_CHUNK,), jnp.float32),
            pltpu.SemaphoreType.DMA,
            pltpu.SemaphoreType.DMA,
        ],
        compiler_params=pltpu.CompilerParams(needs_layout_passes=False),
    )


def _scatter_body(eft_hbm, dst_hbm, agg_hbm, sacc, macc, sacc1, macc1,
                  didx_a, v1_a, v2_a, didx_b, v1_b, v2_b, sem_a, sem_b):
    wid = lax.axis_index("s") * NC + lax.axis_index("c")

    def init_loop(i, carry):
        sl = pl.ds(i * LANES, LANES)
        sacc[sl] = jnp.zeros((LANES,), jnp.float32)
        sacc1[sl] = jnp.zeros((LANES,), jnp.float32)
        macc[sl] = jnp.full((LANES,), -jnp.inf, jnp.float32)
        macc1[sl] = jnp.full((LANES,), -jnp.inf, jnp.float32)
        return carry

    lax.fori_loop(0, N // LANES, init_loop, 0)

    HALF = SE // 2

    def issue(base, bufs, sem):
        bd, b1, b2 = bufs
        half = base // HALF
        loc = base - half * HALF
        v1off = (wid + 64 * half) * HALF + loc
        v2off = (32 + wid + 64 * half) * HALF + loc
        pltpu.async_copy(dst_hbm.at[pl.ds(base, SC_CHUNK)], bd, sem)
        pltpu.async_copy(eft_hbm.at[pl.ds(v1off, SC_CHUNK)], b1, sem)
        pltpu.async_copy(eft_hbm.at[pl.ds(v2off, SC_CHUNK)], b2, sem)

    def drain(bufs, sem):
        bd, b1, b2 = bufs
        pltpu.make_async_copy(dst_hbm.at[pl.ds(0, SC_CHUNK)], bd, sem).wait()
        pltpu.make_async_copy(eft_hbm.at[pl.ds(0, SC_CHUNK)], b1, sem).wait()
        pltpu.make_async_copy(eft_hbm.at[pl.ds(0, SC_CHUNK)], b2, sem).wait()

    def compute(bufs):
        bd, b1, b2 = bufs
        def one_group(off, fail, sa, ma):
            sl = pl.ds(off, LANES)
            d = bd[sl]
            s = b1[sl]
            m = b2[sl]
            plsc.addupdate_scatter(sa, [d], s)
            cur = plsc.load_gather(ma, [d])
            new = jnp.maximum(cur, m)
            plsc.store_scatter(ma, [d], new)
            cur2 = plsc.load_gather(ma, [d])
            plsc.store_scatter(ma, [d], jnp.maximum(cur2, new),
                              mask=cur2 < new)
            chk = plsc.load_gather(ma, [d])
            return fail | (chk < new)

        def grp(gi, fail):
            fail = one_group(gi * (2 * LANES), fail, sacc, macc)
            return one_group(gi * (2 * LANES) + LANES, fail, sacc1, macc1)

        fail = lax.fori_loop(0, NGRP // 2, grp,
                             jnp.zeros((LANES,), jnp.bool_))
        if NGRP % 2:
            fail = one_group((NGRP - 1) * LANES, fail, sacc, macc)

        @pl.when(jnp.any(fail))
        def _fixup():
            def fix_group(off, ma):
                sl = pl.ds(off, LANES)
                d = bd[sl]
                m = b2[sl]

                def retry_body(need):
                    c2 = plsc.load_gather(ma, [d])
                    plsc.store_scatter(ma, [d], jnp.maximum(c2, m),
                                      mask=need)
                    c3 = plsc.load_gather(ma, [d])
                    return c3 < m

                lax.while_loop(lambda n: jnp.any(n), retry_body,
                               plsc.load_gather(ma, [d]) < m)

            def grp2(gi, carry):
                fix_group(gi * (2 * LANES), macc)
                fix_group(gi * (2 * LANES) + LANES, macc1)
                return carry

            lax.fori_loop(0, NGRP // 2, grp2, 0)
            if NGRP % 2:
                fix_group((NGRP - 1) * LANES, macc)

    bufs_a = (didx_a, v1_a, v2_a)
    bufs_b = (didx_b, v1_b, v2_b)
    issue(0, bufs_a, sem_a)

    def pair_loop(ci, carry):
        base_a = (2 * ci) * SC_CHUNK
        base_b = base_a + SC_CHUNK
        base_n = jnp.minimum(base_a + 2 * SC_CHUNK, SE - SC_CHUNK)
        drain(bufs_a, sem_a)
        issue(base_b, bufs_b, sem_b)
        compute(bufs_a)
        drain(bufs_b, sem_b)
        issue(base_n, bufs_a, sem_a)
        compute(bufs_b)
        return carry

    lax.fori_loop(0, NSC // 2, pair_loop, 0)
    drain(bufs_a, sem_a)

    def merge_loop(i, carry):
        sl = pl.ds(i * LANES, LANES)
        sacc[sl] = sacc[sl] + sacc1[sl]
        macc[sl] = jnp.maximum(macc[sl], macc1[sl])
        return carry

    lax.fori_loop(0, N // LANES, merge_loop, 0)

    pltpu.sync_copy(sacc, agg_hbm.at[pl.ds(wid * N, N)])
    pltpu.sync_copy(macc, agg_hbm.at[pl.ds((32 + wid) * N, N)])


def _out_mlp_body(nf_ref, agg0_ref, agg1_ref,
                  wa_ref, wb_ref, wc_ref, b1_ref,
                  w2_ref, b2_ref, w3_ref, b3_ref, w4_ref, b4_ref,
                  out_ref):
    h1 = jnp.dot(nf_ref[...], wa_ref[...], preferred_element_type=jnp.float32)
    sumt = agg0_ref[:32, :] + agg1_ref[:32, :]
    h1 += lax.dot_general(sumt, wb_ref[...], (((0,), (0,)), ((), ())),
                          preferred_element_type=jnp.float32)
    mx = jnp.maximum(agg0_ref[32:, :], agg1_ref[32:, :])
    mx = jnp.where(jnp.isinf(mx) & (mx < 0), 0.0, mx)
    h1 += lax.dot_general(mx, wc_ref[...], (((0,), (0,)), ((), ())),
                          preferred_element_type=jnp.float32)
    h1 = _leaky(h1 + b1_ref[...])
    h2 = _leaky(jnp.dot(h1, w2_ref[...], preferred_element_type=jnp.float32)
                + b2_ref[...])
    h3 = _leaky(jnp.dot(h2, w3_ref[...], preferred_element_type=jnp.float32)
                + b3_ref[...])
    out_ref[...] = jnp.dot(h3, w4_ref[...],
                           preferred_element_type=jnp.float32) + b4_ref[...]


def _out_mlp_call(nf, agg0, agg1, wa, wb, wc, b1, w2, b2, w3, b3, w4, b4):
    return pl.pallas_call(
        _out_mlp_body,
        out_shape=jax.ShapeDtypeStruct((N, OUT_NF), jnp.float32),
    )(nf, agg0, agg1, wa, wb, wc, b1, w2, b2, w3, b3, w4, b4)


@jax.jit
def kernel(nf, ef, msg_params, red_params, edge_index):
    w1, b1, w2, b2, w3, b3, w4, b4 = msg_params
    wr1, br1, wr2, br2, wr3, br3, wr4, br4 = red_params

    src = edge_index[0].astype(jnp.int32)
    dst = edge_index[1].astype(jnp.int32)

    w1s, w1d, w1e = w1[:IN_NF], w1[IN_NF:2 * IN_NF], w1[2 * IN_NF:]
    w4k, w4f = w4[:, :1], w4[:, 1:]
    b4k, b4f = b4[:1].reshape(1, 1), b4[1:].reshape(64, 1)

    def bdiag(w):
        z = jnp.zeros_like(w)
        return jnp.concatenate([jnp.concatenate([w, z], axis=1),
                                jnp.concatenate([z, w], axis=1)], axis=0)

    w1e_bd = bdiag(w1e)
    w2_bd = bdiag(w2)
    w3_bd = bdiag(w3)
    w4k_bd = bdiag(w4k)
    w4f_bd = bdiag(w4f)
    b1p = jnp.tile(b1.reshape(1, 64), (1, 2))
    b2p = jnp.tile(b2.reshape(1, 128), (1, 2))
    b3p = jnp.tile(b3.reshape(1, 64), (1, 2))
    b4k2 = jnp.tile(b4k, (2, 1))
    b4f2 = jnp.tile(b4f, (2, 1))

    p, q = _pq_call(nf, w1s, w1d)
    srcs = [src[s * SE:(s + 1) * SE] for s in range(NSLAB)]
    dsts = [dst[s * SE:(s + 1) * SE] for s in range(NSLAB)]
    dst_cats = [jnp.concatenate([d[0::2], d[1::2]]) for d in dsts]
    efps = [ef[s * SE:(s + 1) * SE].reshape(SE // 2, 2 * IN_EF)
            for s in range(NSLAB)]
    gathered = [_gather_kernel_fn()(p, q, srcs[s], dsts[s])
                for s in range(NSLAB)]
    efts = [_edge_mlp_call(gathered[s][0], gathered[s][1], efps[s],
                           w1e_bd, b1p, w2_bd, b2p, w3_bd, b3p,
                           w4k_bd, b4k2, w4f_bd, b4f2)
            for s in range(NSLAB)]
    aggs = [_scatter_kernel_fn()(efts[s].reshape(64 * SE), dst_cats[s])
            for s in range(NSLAB)]

    wa, wb, wc = wr1[:IN_NF], wr1[IN_NF:IN_NF + 32], wr1[IN_NF + 32:]
    return _out_mlp_call(nf, aggs[0].reshape(64, N), aggs[1].reshape(64, N),
                         wa, wb, wc, br1.reshape(1, 64),
                         wr2, br2.reshape(1, 128), wr3, br3.reshape(1, 64),
                         wr4, br4.reshape(1, OUT_NF))


# speedup vs baseline: 1.4175x; 1.0928x over previous
import functools

import jax
import jax.numpy as jnp
from jax import lax
from jax.experimental import pallas as pl
from jax.experimental.pallas import tpu as pltpu
from jax.experimental.pallas import tpu_sc as plsc

N = 10000
E = 320000
IN_NF = 128
IN_EF = 16
H1 = 32
H2 = 32
OUT_NF = 128

NC, NS, LANES = 2, 16, 16
NW = NC * NS

NSLAB = 2
SE = E // NSLAB
EPW = SE // NW
GC = 128
NGC = EPW // GC
GTAIL = EPW - NGC * GC

SC_CHUNK = 2000
NSC = SE // SC_CHUNK
NGRP = SC_CHUNK // LANES

BE = 1280

def _mesh():
    return plsc.VectorSubcoreMesh(
        core_axis_name="c", subcore_axis_name="s",
        num_cores=NC, num_subcores=NS)


def _leaky(x):
    return jnp.where(x >= 0, x, 0.2 * x)


def _pq_body(nf_ref, ws_ref, wd_ref, p_ref, q_ref):
    nf = nf_ref[...]
    p_ref[...] = jnp.dot(nf, ws_ref[...], preferred_element_type=jnp.float32)
    q_ref[...] = jnp.dot(nf, wd_ref[...], preferred_element_type=jnp.float32)


def _pq_call(nf, w1s, w1d):
    return pl.pallas_call(
        _pq_body,
        out_shape=[jax.ShapeDtypeStruct((N, 64), jnp.float32)] * 2,
    )(nf, w1s, w1d)


@functools.cache
def _gather_kernel_fn():
    return pl.kernel(
        _gather_body,
        out_type=[jax.ShapeDtypeStruct((SE, 64), jnp.float32)] * 2,
        mesh=_mesh(),
        scratch_types=[
            pltpu.VMEM((GC,), jnp.int32),
            pltpu.VMEM((GC,), jnp.int32),
            pltpu.VMEM((GC, 64), jnp.float32),
            pltpu.VMEM((GC, 64), jnp.float32),
            pltpu.VMEM((GC,), jnp.int32),
            pltpu.VMEM((GC,), jnp.int32),
            pltpu.VMEM((GC, 64), jnp.float32),
            pltpu.VMEM((GC, 64), jnp.float32),
            pltpu.SemaphoreType.DMA,
            pltpu.SemaphoreType.DMA,
            pltpu.SemaphoreType.DMA,
            pltpu.SemaphoreType.DMA,
        ],
        compiler_params=pltpu.CompilerParams(use_tc_tiling_on_sc=False),
    )


def _gather_body(p_hbm, q_hbm, src_hbm, dst_hbm, ps_hbm, qd_hbm,
                 sidx_a, didx_a, prow_a, qrow_a,
                 sidx_b, didx_b, prow_b, qrow_b,
                 gsem_a, gsem_b, wsem_a, wsem_b):
    wid = lax.axis_index("s") * NC + lax.axis_index("c")
    base0 = wid * EPW
    A = (sidx_a, didx_a, prow_a, qrow_a, gsem_a, wsem_a)
    B = (sidx_b, didx_b, prow_b, qrow_b, gsem_b, wsem_b)

    def load_and_gather(base, bufs):
        sidx, didx, prow, qrow, gsem, _ = bufs
        pltpu.sync_copy(src_hbm.at[pl.ds(base, GC)], sidx)
        pltpu.sync_copy(dst_hbm.at[pl.ds(base, GC)], didx)
        pltpu.async_copy(p_hbm.at[sidx], prow, gsem)
        pltpu.async_copy(q_hbm.at[didx], qrow, gsem)

    def finish_and_write(base, bufs):
        sidx, didx, prow, qrow, gsem, wsem = bufs
        pltpu.make_async_copy(p_hbm.at[sidx], prow, gsem).wait()
        pltpu.make_async_copy(q_hbm.at[didx], qrow, gsem).wait()
        pltpu.async_copy(prow, ps_hbm.at[pl.ds(base, GC)], wsem)
        pltpu.async_copy(qrow, qd_hbm.at[pl.ds(base, GC)], wsem)

    def drain_writes(bufs):
        _, _, prow, qrow, _, wsem = bufs
        pltpu.make_async_copy(prow, ps_hbm.at[pl.ds(0, GC)], wsem).wait()
        pltpu.make_async_copy(qrow, qd_hbm.at[pl.ds(0, GC)], wsem).wait()

    def pair(ci, carry):
        c0 = base0 + (2 * ci) * GC
        c1 = c0 + GC

        @pl.when(ci > 0)
        def _():
            drain_writes(A)
            drain_writes(B)

        load_and_gather(c0, A)
        load_and_gather(c1, B)
        finish_and_write(c0, A)
        finish_and_write(c1, B)
        return carry

    lax.fori_loop(0, NGC // 2, pair, 0)
    drain_writes(A)
    drain_writes(B)

    def do_chunk(base, n):
        sidx, didx, prow, qrow, gsem, _ = A
        pltpu.sync_copy(src_hbm.at[pl.ds(base, n)], sidx.at[pl.ds(0, n)])
        pltpu.sync_copy(dst_hbm.at[pl.ds(base, n)], didx.at[pl.ds(0, n)])
        cp1 = pltpu.async_copy(p_hbm.at[sidx.at[pl.ds(0, n)]],
                               prow.at[pl.ds(0, n)], gsem)
        cp2 = pltpu.async_copy(q_hbm.at[didx.at[pl.ds(0, n)]],
                               qrow.at[pl.ds(0, n)], gsem)
        cp1.wait()
        cp2.wait()
        pltpu.sync_copy(prow.at[pl.ds(0, n)], ps_hbm.at[pl.ds(base, n)])
        pltpu.sync_copy(qrow.at[pl.ds(0, n)], qd_hbm.at[pl.ds(base, n)])

    if NGC % 2:
        do_chunk(base0 + (NGC - 1) * GC, GC)
    if GTAIL:
        do_chunk(base0 + NGC * GC, GTAIL)


def _edge_mlp_body(ps_ref, qd_ref, efp_ref,
                   w1e_ref, b1_ref, w2_ref, b2_ref, w3_ref, b3_ref,
                   w4k_ref, b4k_ref, w4f_ref, b4f_ref,
                   out_ref):
    x = ps_ref[...] + qd_ref[...]
    x += jnp.dot(efp_ref[...], w1e_ref[...], preferred_element_type=jnp.float32)
    h1 = _leaky(x + b1_ref[...])
    h2 = _leaky(jnp.dot(h1, w2_ref[...], preferred_element_type=jnp.float32)
                + b2_ref[...])
    h3 = _leaky(jnp.dot(h2, w3_ref[...], preferred_element_type=jnp.float32)
                + b3_ref[...])
    kT = lax.dot_general(w4k_ref[...], h3, (((0,), (1,)), ((), ())),
                         preferred_element_type=jnp.float32)
    kT = 1.0 / (1.0 + jnp.exp(-(kT + b4k_ref[...])))
    fT = lax.dot_general(w4f_ref[...], h3, (((0,), (1,)), ((), ())),
                         preferred_element_type=jnp.float32)
    fT = fT + b4f_ref[...]
    out_ref[0:64, :] = fT[0:64, :] * kT[0:1, :]
    out_ref[64:128, :] = fT[64:128, :] * kT[1:2, :]


BE2 = BE // 2


def _edge_mlp_call(ps, qd, efp, slab,
                   w1e, b1, w2, b2, w3, b3, w4k, b4k, w4f, b4f):
    grid = SE // BE
    soff = slab * grid
    wspec = lambda shape: pl.BlockSpec(shape, lambda i: (0,) * len(shape))
    return pl.pallas_call(
        _edge_mlp_body,
        grid=(grid,),
        in_specs=[
            pl.BlockSpec((BE2, 128), lambda i: (i, 0)),
            pl.BlockSpec((BE2, 128), lambda i: (i, 0)),
            pl.BlockSpec((BE2, 2 * IN_EF), lambda i: (i + soff, 0)),
            wspec((2 * IN_EF, 128)), wspec((1, 128)),
            wspec((128, 256)), wspec((1, 256)),
            wspec((256, 128)), wspec((1, 128)),
            wspec((128, 2)), wspec((2, 1)),
            wspec((128, 128)), wspec((128, 1)),
        ],
        out_specs=pl.BlockSpec((128, BE2), lambda i: (0, i)),
        out_shape=jax.ShapeDtypeStruct((128, SE // 2), jnp.float32),
    )(ps.reshape(SE // 2, 128), qd.reshape(SE // 2, 128), efp,
      w1e, b1, w2, b2, w3, b3, w4k, b4k, w4f, b4f)


@functools.cache
def _scatter_kernel_fn():
    return pl.kernel(
        _scatter_body,
        out_type=jax.ShapeDtypeStruct((64 * N,), jnp.float32),
        mesh=_mesh(),
        scratch_types=[
            pltpu.VMEM((N,), jnp.float32),
            pltpu.VMEM((N,), jnp.float32),
            pltpu.VMEM((N,), jnp.float32),
            pltpu.VMEM((N,), jnp.float32),
            pltpu.VMEM((SC_CHUNK,), jnp.int32),
            pltpu.VMEM((SC_CHUNK,), jnp.float32),
            pltpu.VMEM((SC_CHUNK,), jnp.float32),
            pltpu.VMEM((SC_CHUNK,), jnp.int32),
            pltpu.VMEM((SC_CHUNK,), jnp.float32),
            pltpu.VMEM((SC_CHUNK,), jnp.float32),
            pltpu.SemaphoreType.DMA,
            pltpu.SemaphoreType.DMA,
        ],
        compiler_params=pltpu.CompilerParams(needs_layout_passes=False),
    )


def _scatter_body(eft_hbm, dst_hbm, agg_hbm, sacc, macc, sacc1, macc1,
                  didx_a, v1_a, v2_a, didx_b, v1_b, v2_b, sem_a, sem_b):
    wid = lax.axis_index("s") * NC + lax.axis_index("c")

    def init_loop(i, carry):
        sl = pl.ds(i * LANES, LANES)
        sacc[sl] = jnp.zeros((LANES,), jnp.float32)
        sacc1[sl] = jnp.zeros((LANES,), jnp.float32)
        macc[sl] = jnp.full((LANES,), -jnp.inf, jnp.float32)
        macc1[sl] = jnp.full((LANES,), -jnp.inf, jnp.float32)
        return carry

    lax.fori_loop(0, N // LANES, init_loop, 0)

    HALF = SE // 2

    def issue(base, bufs, sem):
        bd, b1, b2 = bufs
        half = base // HALF
        loc = base - half * HALF
        v1off = (wid + 64 * half) * HALF + loc
        v2off = (32 + wid + 64 * half) * HALF + loc
        pltpu.async_copy(dst_hbm.at[pl.ds(base, SC_CHUNK)], bd, sem)
        pltpu.async_copy(eft_hbm.at[pl.ds(v1off, SC_CHUNK)], b1, sem)
        pltpu.async_copy(eft_hbm.at[pl.ds(v2off, SC_CHUNK)], b2, sem)

    def drain(bufs, sem):
        bd, b1, b2 = bufs
        pltpu.make_async_copy(dst_hbm.at[pl.ds(0, SC_CHUNK)], bd, sem).wait()
        pltpu.make_async_copy(eft_hbm.at[pl.ds(0, SC_CHUNK)], b1, sem).wait()
        pltpu.make_async_copy(eft_hbm.at[pl.ds(0, SC_CHUNK)], b2, sem).wait()

    def compute(bufs):
        bd, b1, b2 = bufs
        def one_group(off, fail, sa, ma):
            sl = pl.ds(off, LANES)
            d = bd[sl]
            s = b1[sl]
            m = b2[sl]
            plsc.addupdate_scatter(sa, [d], s)
            cur = plsc.load_gather(ma, [d])
            new = jnp.maximum(cur, m)
            plsc.store_scatter(ma, [d], new)
            cur2 = plsc.load_gather(ma, [d])
            plsc.store_scatter(ma, [d], jnp.maximum(cur2, new),
                              mask=cur2 < new)
            chk = plsc.load_gather(ma, [d])
            return fail | (chk < new)

        def grp(gi, fail):
            fail = one_group(gi * (2 * LANES), fail, sacc, macc)
            return one_group(gi * (2 * LANES) + LANES, fail, sacc1, macc1)

        fail = lax.fori_loop(0, NGRP // 2, grp,
                             jnp.zeros((LANES,), jnp.bool_))
        if NGRP % 2:
            fail = one_group((NGRP - 1) * LANES, fail, sacc, macc)

        @pl.when(jnp.any(fail))
        def _fixup():
            def fix_group(off, ma):
                sl = pl.ds(off, LANES)
                d = bd[sl]
                m = b2[sl]

                def retry_body(need):
                    c2 = plsc.load_gather(ma, [d])
                    plsc.store_scatter(ma, [d], jnp.maximum(c2, m),
                                      mask=need)
                    c3 = plsc.load_gather(ma, [d])
                    return c3 < m

                lax.while_loop(lambda n: jnp.any(n), retry_body,
                               plsc.load_gather(ma, [d]) < m)

            def grp2(gi, carry):
                fix_group(gi * (2 * LANES), macc)
                fix_group(gi * (2 * LANES) + LANES, macc1)
                return carry

            lax.fori_loop(0, NGRP // 2, grp2, 0)
            if NGRP % 2:
                fix_group((NGRP - 1) * LANES, macc)

    bufs_a = (didx_a, v1_a, v2_a)
    bufs_b = (didx_b, v1_b, v2_b)
    issue(0, bufs_a, sem_a)

    def pair_loop(ci, carry):
        base_a = (2 * ci) * SC_CHUNK
        base_b = base_a + SC_CHUNK
        base_n = jnp.minimum(base_a + 2 * SC_CHUNK, SE - SC_CHUNK)
        drain(bufs_a, sem_a)
        issue(base_b, bufs_b, sem_b)
        compute(bufs_a)
        drain(bufs_b, sem_b)
        issue(base_n, bufs_a, sem_a)
        compute(bufs_b)
        return carry

    lax.fori_loop(0, NSC // 2, pair_loop, 0)
    drain(bufs_a, sem_a)

    def merge_loop(i, carry):
        sl = pl.ds(i * LANES, LANES)
        sacc[sl] = sacc[sl] + sacc1[sl]
        macc[sl] = jnp.maximum(macc[sl], macc1[sl])
        return carry

    lax.fori_loop(0, N // LANES, merge_loop, 0)

    pltpu.sync_copy(sacc, agg_hbm.at[pl.ds(wid * N, N)])
    pltpu.sync_copy(macc, agg_hbm.at[pl.ds((32 + wid) * N, N)])


def _out_mlp_body(nf_ref, agg0_ref, agg1_ref,
                  wa_ref, wb_ref, wc_ref, b1_ref,
                  w2_ref, b2_ref, w3_ref, b3_ref, w4_ref, b4_ref,
                  out_ref):
    h1 = jnp.dot(nf_ref[...], wa_ref[...], preferred_element_type=jnp.float32)
    sumt = agg0_ref[:32, :] + agg1_ref[:32, :]
    h1 += lax.dot_general(sumt, wb_ref[...], (((0,), (0,)), ((), ())),
                          preferred_element_type=jnp.float32)
    mx = jnp.maximum(agg0_ref[32:, :], agg1_ref[32:, :])
    mx = jnp.where(jnp.isinf(mx) & (mx < 0), 0.0, mx)
    h1 += lax.dot_general(mx, wc_ref[...], (((0,), (0,)), ((), ())),
                          preferred_element_type=jnp.float32)
    h1 = _leaky(h1 + b1_ref[...])
    h2 = _leaky(jnp.dot(h1, w2_ref[...], preferred_element_type=jnp.float32)
                + b2_ref[...])
    h3 = _leaky(jnp.dot(h2, w3_ref[...], preferred_element_type=jnp.float32)
                + b3_ref[...])
    out_ref[...] = jnp.dot(h3, w4_ref[...],
                           preferred_element_type=jnp.float32) + b4_ref[...]


def _out_mlp_call(nf, agg0, agg1, wa, wb, wc, b1, w2, b2, w3, b3, w4, b4):
    return pl.pallas_call(
        _out_mlp_body,
        out_shape=jax.ShapeDtypeStruct((N, OUT_NF), jnp.float32),
    )(nf, agg0, agg1, wa, wb, wc, b1, w2, b2, w3, b3, w4, b4)


@jax.jit
def kernel(nf, ef, msg_params, red_params, edge_index):
    w1, b1, w2, b2, w3, b3, w4, b4 = msg_params
    wr1, br1, wr2, br2, wr3, br3, wr4, br4 = red_params

    src = edge_index[0].astype(jnp.int32)
    dst = edge_index[1].astype(jnp.int32)

    w1s, w1d, w1e = w1[:IN_NF], w1[IN_NF:2 * IN_NF], w1[2 * IN_NF:]
    w4k, w4f = w4[:, :1], w4[:, 1:]
    b4k, b4f = b4[:1].reshape(1, 1), b4[1:].reshape(64, 1)

    def bdiag(w):
        z = jnp.zeros_like(w)
        return jnp.concatenate([jnp.concatenate([w, z], axis=1),
                                jnp.concatenate([z, w], axis=1)], axis=0)

    w1e_bd = bdiag(w1e)
    w2_bd = bdiag(w2)
    w3_bd = bdiag(w3)
    w4k_bd = bdiag(w4k)
    w4f_bd = bdiag(w4f)
    b1p = jnp.tile(b1.reshape(1, 64), (1, 2))
    b2p = jnp.tile(b2.reshape(1, 128), (1, 2))
    b3p = jnp.tile(b3.reshape(1, 64), (1, 2))
    b4k2 = jnp.tile(b4k, (2, 1))
    b4f2 = jnp.tile(b4f, (2, 1))

    p, q = _pq_call(nf, w1s, w1d)
    srcs = [src[s * SE:(s + 1) * SE] for s in range(NSLAB)]
    dsts = [dst[s * SE:(s + 1) * SE] for s in range(NSLAB)]
    dst_cats = [jnp.concatenate([d[0::2], d[1::2]]) for d in dsts]
    efp = ef.reshape(E // 2, 2 * IN_EF)
    gathered = [_gather_kernel_fn()(p, q, srcs[s], dsts[s])
                for s in range(NSLAB)]
    efts = [_edge_mlp_call(gathered[s][0], gathered[s][1], efp, s,
                           w1e_bd, b1p, w2_bd, b2p, w3_bd, b3p,
                           w4k_bd, b4k2, w4f_bd, b4f2)
            for s in range(NSLAB)]
    aggs = [_scatter_kernel_fn()(efts[s].reshape(64 * SE), dst_cats[s])
            for s in range(NSLAB)]

    wa, wb, wc = wr1[:IN_NF], wr1[IN_NF:IN_NF + 32], wr1[IN_NF + 32:]
    return _out_mlp_call(nf, aggs[0].reshape(64, N), aggs[1].reshape(64, N),
                         wa, wb, wc, br1.reshape(1, 64),
                         wr2, br2.reshape(1, 128), wr3, br3.reshape(1, 64),
                         wr4, br4.reshape(1, OUT_NF))
